# baseline (speedup 1.0000x reference)
import jax
import jax.numpy as jnp
from jax.experimental import pallas as pl
from jax.experimental.pallas import tpu as pltpu

_NBUF = 48
_LAG = 24


def _pack_body(sel_ref, slot_ref, in_ref, fast_ref, slow_ref, buf, in_sems,
               out_sems, slow_sems):
    T = in_ref.shape[1]
    in_descs = [None] * T
    out_descs = [None] * T
    slow_descs = [None] * T

    def slow_start(t, b):
        slow_descs[t] = pltpu.make_async_copy(
            buf.at[b], slow_ref.at[:, pl.ds(slot_ref[t], 1)], slow_sems.at[b]
        )

        @pl.when(sel_ref[t] != 0)
        def _():
            slow_descs[t].start()

    def slow_wait(t):
        @pl.when(sel_ref[t] != 0)
        def _():
            slow_descs[t].wait()

    for i in range(T + _LAG):
        if i < T:
            b = i % _NBUF
            if i >= _NBUF:
                out_descs[i - _NBUF].wait()
                slow_wait(i - _NBUF)
            in_descs[i] = pltpu.make_async_copy(
                in_ref.at[:, pl.ds(i, 1)], buf.at[b], in_sems.at[b]
            )
            in_descs[i].start()
        j = i - _LAG
        if 0 <= j < T:
            bj = j % _NBUF
            in_descs[j].wait()
            out_descs[j] = pltpu.make_async_copy(
                buf.at[bj], fast_ref.at[:, pl.ds(j, 1)], out_sems.at[bj]
            )
            out_descs[j].start()
            slow_start(j, bj)
    for j in range(T - _NBUF, T):
        out_descs[j].wait()
        slow_wait(j)


def kernel(frames, slowfast_alpha):
    del slowfast_alpha
    C, T, H, W = frames.shape
    num = T // 4
    idx = jnp.linspace(0.0, T - 1, num).astype(jnp.int32)
    t_range = jnp.arange(T, dtype=jnp.int32)
    slot = jnp.searchsorted(idx, t_range, side="right").astype(jnp.int32) - 1
    slot = jnp.clip(slot, 0, num - 1)
    sel = (jnp.take(idx, slot) == t_range).astype(jnp.int32)

    grid_spec = pltpu.PrefetchScalarGridSpec(
        num_scalar_prefetch=2,
        grid=(1,),
        in_specs=[pl.BlockSpec(memory_space=pltpu.MemorySpace.HBM)],
        out_specs=[
            pl.BlockSpec(memory_space=pltpu.MemorySpace.HBM),
            pl.BlockSpec(memory_space=pltpu.MemorySpace.HBM),
        ],
        scratch_shapes=[
            pltpu.VMEM((_NBUF, C, 1, H, W), frames.dtype),
            pltpu.SemaphoreType.DMA((_NBUF,)),
            pltpu.SemaphoreType.DMA((_NBUF,)),
            pltpu.SemaphoreType.DMA((_NBUF,)),
        ],
    )
    fast, slow = pl.pallas_call(
        _pack_body,
        grid_spec=grid_spec,
        out_shape=[
            jax.ShapeDtypeStruct((C, T, H, W), frames.dtype),
            jax.ShapeDtypeStruct((C, num, H, W), frames.dtype),
        ],
    )(sel, slot, frames)
    return (slow, fast)


# speedup vs baseline: 1.0108x; 1.0108x over previous
import jax
import jax.numpy as jnp
from jax.experimental import pallas as pl
from jax.experimental.pallas import tpu as pltpu

_NBUF = 32
_LAG = 16


def _pack_body(sel_ref, slot_ref, in_ref, fast_ref, slow_ref, buf, in_sems,
               out_sems, slow_sems):
    T = in_ref.shape[1]
    in_descs = [None] * T
    out_descs = [None] * T
    slow_descs = [None] * T

    def slow_start(t, b):
        slow_descs[t] = pltpu.make_async_copy(
            buf.at[b], slow_ref.at[:, pl.ds(slot_ref[t], 1)], slow_sems.at[b]
        )

        @pl.when(sel_ref[t] != 0)
        def _():
            slow_descs[t].start()

    def slow_wait(t):
        @pl.when(sel_ref[t] != 0)
        def _():
            slow_descs[t].wait()

    for i in range(T + _LAG):
        if i < T:
            b = i % _NBUF
            if i >= _NBUF:
                out_descs[i - _NBUF].wait()
                slow_wait(i - _NBUF)
            in_descs[i] = pltpu.make_async_copy(
                in_ref.at[:, pl.ds(i, 1)], buf.at[b], in_sems.at[b]
            )
            in_descs[i].start()
        j = i - _LAG
        if 0 <= j < T:
            bj = j % _NBUF
            in_descs[j].wait()
            out_descs[j] = pltpu.make_async_copy(
                buf.at[bj], fast_ref.at[:, pl.ds(j, 1)], out_sems.at[bj]
            )
            out_descs[j].start()
            slow_start(j, bj)
    for j in range(T - _NBUF, T):
        out_descs[j].wait()
        slow_wait(j)


def kernel(frames, slowfast_alpha):
    del slowfast_alpha
    C, T, H, W = frames.shape
    num = T // 4
    idx = jnp.linspace(0.0, T - 1, num).astype(jnp.int32)
    t_range = jnp.arange(T, dtype=jnp.int32)
    slot = jnp.searchsorted(idx, t_range, side="right").astype(jnp.int32) - 1
    slot = jnp.clip(slot, 0, num - 1)
    sel = (jnp.take(idx, slot) == t_range).astype(jnp.int32)

    grid_spec = pltpu.PrefetchScalarGridSpec(
        num_scalar_prefetch=2,
        grid=(1,),
        in_specs=[pl.BlockSpec(memory_space=pltpu.MemorySpace.HBM)],
        out_specs=[
            pl.BlockSpec(memory_space=pltpu.MemorySpace.HBM),
            pl.BlockSpec(memory_space=pltpu.MemorySpace.HBM),
        ],
        scratch_shapes=[
            pltpu.VMEM((_NBUF, C, 1, H, W), frames.dtype),
            pltpu.SemaphoreType.DMA((_NBUF,)),
            pltpu.SemaphoreType.DMA((_NBUF,)),
            pltpu.SemaphoreType.DMA((_NBUF,)),
        ],
    )
    fast, slow = pl.pallas_call(
        _pack_body,
        grid_spec=grid_spec,
        out_shape=[
            jax.ShapeDtypeStruct((C, T, H, W), frames.dtype),
            jax.ShapeDtypeStruct((C, num, H, W), frames.dtype),
        ],
    )(sel, slot, frames)
    return (slow, fast)


# speedup vs baseline: 1.0215x; 1.0107x over previous
import jax
import jax.numpy as jnp
from jax.experimental import pallas as pl
from jax.experimental.pallas import tpu as pltpu

_NBUF = 16
_LAG = 8


def _pack_body(sel_ref, off_ref, slot_ref, in_ref, fast_ref, slow_ref, buf,
               in_sems, out_sems, slow_sems):
    U = in_ref.shape[1] // 2
    in_descs = [None] * U
    out_descs = [None] * U
    slow_descs = [None] * U

    def slow_start(u, b):
        slow_descs[u] = pltpu.make_async_copy(
            buf.at[b, :, pl.ds(off_ref[u], 1)],
            slow_ref.at[:, pl.ds(slot_ref[u], 1)],
            slow_sems.at[b],
        )

        @pl.when(sel_ref[u] != 0)
        def _():
            slow_descs[u].start()

    def slow_wait(u):
        @pl.when(sel_ref[u] != 0)
        def _():
            slow_descs[u].wait()

    for i in range(U + _LAG):
        if i < U:
            b = i % _NBUF
            if i >= _NBUF:
                out_descs[i - _NBUF].wait()
                slow_wait(i - _NBUF)
            in_descs[i] = pltpu.make_async_copy(
                in_ref.at[:, pl.ds(2 * i, 2)], buf.at[b], in_sems.at[b]
            )
            in_descs[i].start()
        j = i - _LAG
        if 0 <= j < U:
            bj = j % _NBUF
            in_descs[j].wait()
            out_descs[j] = pltpu.make_async_copy(
                buf.at[bj], fast_ref.at[:, pl.ds(2 * j, 2)], out_sems.at[bj]
            )
            out_descs[j].start()
            slow_start(j, bj)
    for j in range(U - _NBUF, U):
        out_descs[j].wait()
        slow_wait(j)


def kernel(frames, slowfast_alpha):
    del slowfast_alpha
    C, T, H, W = frames.shape
    num = T // 4
    idx = jnp.linspace(0.0, T - 1, num).astype(jnp.int32)
    t_range = jnp.arange(T, dtype=jnp.int32)
    slot = jnp.searchsorted(idx, t_range, side="right").astype(jnp.int32) - 1
    slot = jnp.clip(slot, 0, num - 1)
    sel = (jnp.take(idx, slot) == t_range).astype(jnp.int32)
    sel_e, sel_o = sel[0::2], sel[1::2]
    sel_u = sel_e | sel_o
    off_u = sel_o
    slot_u = sel_e * slot[0::2] + sel_o * slot[1::2]

    grid_spec = pltpu.PrefetchScalarGridSpec(
        num_scalar_prefetch=3,
        grid=(1,),
        in_specs=[pl.BlockSpec(memory_space=pltpu.MemorySpace.HBM)],
        out_specs=[
            pl.BlockSpec(memory_space=pltpu.MemorySpace.HBM),
            pl.BlockSpec(memory_space=pltpu.MemorySpace.HBM),
        ],
        scratch_shapes=[
            pltpu.VMEM((_NBUF, C, 2, H, W), frames.dtype),
            pltpu.SemaphoreType.DMA((_NBUF,)),
            pltpu.SemaphoreType.DMA((_NBUF,)),
            pltpu.SemaphoreType.DMA((_NBUF,)),
        ],
    )
    fast, slow = pl.pallas_call(
        _pack_body,
        grid_spec=grid_spec,
        out_shape=[
            jax.ShapeDtypeStruct((C, T, H, W), frames.dtype),
            jax.ShapeDtypeStruct((C, num, H, W), frames.dtype),
        ],
    )(sel_u, off_u, slot_u, frames)
    return (slow, fast)
